# Initial kernel scaffold; baseline (speedup 1.0000x reference)
import jax
import jax.numpy as jnp
from jax.experimental import pallas as pl


def kernel(x, edge_index, degrees, W1, b1, W2, b2, eps):
    raise NotImplementedError("write your pallas kernel here")


# speedup vs baseline: 7.1962x; 7.1962x over previous
import functools

import jax
import jax.numpy as jnp
from jax import lax
from jax.experimental import pallas as pl
from jax.experimental.pallas import tpu as pltpu
from jax.experimental.pallas import tpu_sc as plsc

N = 10000
E = 320000
D = 128

NC = 2
NS = 16
NW = NC * NS

EPW = E // NW
CHUNK = 80
NCHUNK = EPW // CHUNK

N_ACC = 10240
STRIPE = N_ACC // NS


def _sc_aggregate(x, src, dst):
    mesh = plsc.VectorSubcoreMesh(
        core_axis_name="c", subcore_axis_name="s", num_cores=NC, num_subcores=NS
    )

    @functools.partial(
        pl.kernel,
        out_type=jax.ShapeDtypeStruct((NC, N_ACC, D), jnp.float32),
        mesh=mesh,
        scratch_types=[
            pltpu.VMEM((NCHUNK, CHUNK), jnp.int32),
            pltpu.VMEM((NCHUNK, CHUNK), jnp.int32),
            pltpu.VMEM((CHUNK, D), jnp.float32),
            pltpu.VMEM_SHARED((N_ACC, D), jnp.float32),
            pltpu.SemaphoreType.DMA,
        ],
    )
    def k(x_hbm, src_hbm, dst_hbm, out_hbm, src_v, dst_v, buf, acc, sem):
        cid = lax.axis_index("c")
        sid = lax.axis_index("s")
        wid = cid * NS + sid

        zero = jnp.zeros((16,), jnp.float32)

        def zr(i, carry):
            buf[i // 8, pl.ds((i % 8) * 16, 16)] = zero
            return carry

        lax.fori_loop(0, CHUNK * 8, zr, 0)
        for r in range(STRIPE // CHUNK):
            pltpu.sync_copy(buf, acc.at[pl.ds(sid * STRIPE + r * CHUNK, CHUNK)])

        pltpu.sync_copy(src_hbm.at[wid], src_v)
        pltpu.sync_copy(dst_hbm.at[wid], dst_v)
        plsc.subcore_barrier()

        def body(c, carry):
            pltpu.async_copy(x_hbm.at[src_v.at[c]], buf, sem).wait()
            pltpu.sync_copy(buf, acc.at[dst_v.at[c]], add=True)
            return carry

        lax.fori_loop(0, NCHUNK, body, 0)

        plsc.subcore_barrier()
        pltpu.sync_copy(
            acc.at[pl.ds(sid * STRIPE, STRIPE)],
            out_hbm.at[cid, pl.ds(sid * STRIPE, STRIPE)],
        )

    return k(x, src, dst)


ROWS = 400


def _mlp_body(x_ref, p0_ref, p1_ref, w1_ref, b1_ref, w2_ref, b2_ref, eps_ref, o_ref):
    h = (1.0 + eps_ref[0, 0]) * x_ref[...] + p0_ref[...] + p1_ref[...]
    h = jnp.dot(h, w1_ref[...], preferred_element_type=jnp.float32) + b1_ref[...]
    h = jnp.maximum(h, 0.0)
    o_ref[...] = jnp.dot(h, w2_ref[...], preferred_element_type=jnp.float32) + b2_ref[...]


def _mlp(x, p0, p1, W1, b1, W2, b2, eps):
    grid = (N // ROWS,)
    return pl.pallas_call(
        _mlp_body,
        grid=grid,
        in_specs=[
            pl.BlockSpec((ROWS, D), lambda i: (i, 0)),
            pl.BlockSpec((ROWS, D), lambda i: (i, 0)),
            pl.BlockSpec((ROWS, D), lambda i: (i, 0)),
            pl.BlockSpec((D, D), lambda i: (0, 0)),
            pl.BlockSpec((1, D), lambda i: (0, 0)),
            pl.BlockSpec((D, D), lambda i: (0, 0)),
            pl.BlockSpec((1, D), lambda i: (0, 0)),
            pl.BlockSpec(memory_space=pltpu.SMEM),
        ],
        out_specs=pl.BlockSpec((ROWS, D), lambda i: (i, 0)),
        out_shape=jax.ShapeDtypeStruct((N, D), jnp.float32),
    )(x, p0, p1, W1, b1.reshape(1, D), W2, b2.reshape(1, D), eps.reshape(1, 1))


def kernel(x, edge_index, degrees, W1, b1, W2, b2, eps):
    src = edge_index[0].reshape(NW, NCHUNK, CHUNK)
    dst = edge_index[1].reshape(NW, NCHUNK, CHUNK)
    partial = _sc_aggregate(x, src, dst)
    return _mlp(x, partial[0, :N], partial[1, :N], W1, b1, W2, b2, eps)
